# baseline (device time: 31327 ns/iter reference)
import jax
import jax.numpy as jnp
from jax import lax
from jax.experimental import pallas as pl
from jax.experimental.pallas import tpu as pltpu

N_DEV = 4
BLK = 512


def kernel(x, dy, gamma):
    m, d = x.shape
    nblk = m // BLK

    def body(x_ref, dy_ref, out_ref, comm_ref, send_sems, recv_sems):
        i = pl.program_id(0)

        xb = x_ref[...]
        dyb = dy_ref[...]
        mu = jnp.mean(xb, axis=1, keepdims=True)
        xc = xb - mu
        var = jnp.mean(xc * xc, axis=1, keepdims=True)
        rstd = lax.rsqrt(var + 1e-5)
        xhat = xc * rstd
        dgamma = jnp.sum(dyb * xhat, axis=0)
        dbeta = jnp.sum(dyb, axis=0)
        partial = jnp.stack([dgamma, dbeta])

        @pl.when(i == 0)
        def _():
            out_ref[...] = partial

        @pl.when(i != 0)
        def _():
            out_ref[...] += partial

        @pl.when(i == nblk - 1)
        def _():
            my = lax.axis_index("i")
            left = (my + N_DEV - 1) % N_DEV
            right = (my + 1) % N_DEV

            barrier = pltpu.get_barrier_semaphore()
            for nbr in (left, right):
                pl.semaphore_signal(
                    barrier, inc=1,
                    device_id=(nbr,), device_id_type=pl.DeviceIdType.MESH,
                )
            pl.semaphore_wait(barrier, 2)

            comm_ref[0, :, :] = out_ref[...]
            for h in range(N_DEV - 1):
                rdma = pltpu.make_async_remote_copy(
                    src_ref=comm_ref.at[h],
                    dst_ref=comm_ref.at[h + 1],
                    send_sem=send_sems.at[h],
                    recv_sem=recv_sems.at[h],
                    device_id=(right,),
                    device_id_type=pl.DeviceIdType.MESH,
                )
                rdma.start()
                rdma.wait()
                out_ref[...] += comm_ref[h + 1, :, :]

    return pl.pallas_call(
        body,
        grid=(nblk,),
        in_specs=[
            pl.BlockSpec((BLK, d), lambda i: (i, 0)),
            pl.BlockSpec((BLK, d), lambda i: (i, 0)),
        ],
        out_specs=pl.BlockSpec((2, d), lambda i: (0, 0)),
        out_shape=jax.ShapeDtypeStruct((2, d), jnp.float32),
        scratch_shapes=[
            pltpu.VMEM((N_DEV, 2, d), jnp.float32),
            pltpu.SemaphoreType.DMA((N_DEV - 1,)),
            pltpu.SemaphoreType.DMA((N_DEV - 1,)),
        ],
        compiler_params=pltpu.CompilerParams(collective_id=0),
    )(x, dy)


# device time: 29321 ns/iter; 1.0684x vs baseline; 1.0684x over previous
import jax
import jax.numpy as jnp
from jax import lax
from jax.experimental import pallas as pl
from jax.experimental.pallas import tpu as pltpu

N_DEV = 4
BLK = 512


def kernel(x, dy, gamma):
    m, d = x.shape
    nblk = m // BLK

    def body(x_ref, dy_ref, out_ref, comm_ref, send_sems, recv_sems):
        i = pl.program_id(0)
        my = lax.axis_index("i")

        @pl.when(i == 0)
        def _():
            barrier = pltpu.get_barrier_semaphore()
            for k in range(1, N_DEV):
                pl.semaphore_signal(
                    barrier, inc=1, device_id=((my + k) % N_DEV,),
                    device_id_type=pl.DeviceIdType.MESH)
            pl.semaphore_wait(barrier, N_DEV - 1)

        xb = x_ref[...]
        dyb = dy_ref[...]
        mu = jnp.mean(xb, axis=1, keepdims=True)
        var = jnp.mean(xb * xb, axis=1, keepdims=True) - mu * mu
        rstd = lax.rsqrt(var + 1e-5)
        b_ = -rstd * mu
        t1 = lax.dot_general(rstd, dyb * xb, (((0,), (0,)), ((), ())),
                             preferred_element_type=jnp.float32)
        W = jnp.concatenate([b_, jnp.ones_like(b_)], axis=1)
        t2 = lax.dot_general(W, dyb, (((0,), (0,)), ((), ())),
                             preferred_element_type=jnp.float32)
        partial = jnp.stack([t1[0] + t2[0], t2[1]])

        @pl.when(i == 0)
        def _():
            out_ref[...] = partial

        @pl.when(i != 0)
        def _():
            out_ref[...] += partial

        @pl.when(i == nblk - 1)
        def _():
            rdmas = []
            for k in range(1, N_DEV):
                rdma = pltpu.make_async_remote_copy(
                    src_ref=out_ref, dst_ref=comm_ref.at[k - 1],
                    send_sem=send_sems.at[k - 1],
                    recv_sem=recv_sems.at[k - 1],
                    device_id=((my + k) % N_DEV,),
                    device_id_type=pl.DeviceIdType.MESH)
                rdma.start()
                rdmas.append(rdma)
            for r in rdmas:
                r.wait_send()
            for r in rdmas:
                r.wait_recv()
            out_ref[...] += (comm_ref[0, :, :] + comm_ref[1, :, :]
                             ) + comm_ref[2, :, :]

    return pl.pallas_call(
        body,
        grid=(nblk,),
        in_specs=[pl.BlockSpec((BLK, d), lambda i: (i, 0)),
                  pl.BlockSpec((BLK, d), lambda i: (i, 0))],
        out_specs=pl.BlockSpec((2, d), lambda i: (0, 0)),
        out_shape=jax.ShapeDtypeStruct((2, d), jnp.float32),
        scratch_shapes=[
            pltpu.VMEM((N_DEV - 1, 2, d), jnp.float32),
            pltpu.SemaphoreType.DMA((N_DEV - 1,)),
            pltpu.SemaphoreType.DMA((N_DEV - 1,)),
        ],
        compiler_params=pltpu.CompilerParams(
            collective_id=0, vmem_limit_bytes=64 * 1024 * 1024),
    )(x, dy)
